# baseline (speedup 1.0000x reference)
import functools

import jax
import jax.numpy as jnp
from jax import lax
from jax.experimental import pallas as pl
from jax.experimental.pallas import tpu as pltpu
from jax.experimental.pallas import tpu_sc as plsc

_N, _D, _L, _NB, _NR, _B, _E = 10000, 128, 3, 4, 64, 512, 160000
_BD = _NB * _D

_NC, _NS, _LN = 2, 16, 16
_NW = _NC * _NS

_EP = 163840
_EPW = _EP // _NW
_ECH = 64
_ENCH = _EPW // _ECH
_NPE = 10112
_ZR = _NPE // _NS

_NP2 = 10240
_RPW = _NP2 // _NW
_PCH = 64
_PNCH = _RPW // _PCH
_GP = 640
_GZR = _GP // _NS
_BW = _B // _NW

_RB = 400


def _mesh():
    return plsc.VectorSubcoreMesh(core_axis_name="c", subcore_axis_name="s")


def _mm_first_body(x_ref, w_ref, b_ref, y_ref, s_ref):
    p = jnp.dot(x_ref[...], w_ref[...], preferred_element_type=jnp.float32)
    y_ref[...] = p[:, :_BD]
    s_ref[...] = p[:, _BD:] + b_ref[...]


def _mm_first(x, wcat, b):
    return pl.pallas_call(
        _mm_first_body,
        grid=(_N // _RB,),
        in_specs=[
            pl.BlockSpec((_RB, _D), lambda i: (i, 0)),
            pl.BlockSpec((_D, _BD + _D), lambda i: (0, 0)),
            pl.BlockSpec((1, _D), lambda i: (0, 0)),
        ],
        out_specs=[
            pl.BlockSpec((_RB, _BD), lambda i: (i, 0)),
            pl.BlockSpec((_RB, _D), lambda i: (i, 0)),
        ],
        out_shape=[
            jax.ShapeDtypeStruct((_N, _BD), jnp.float32),
            jax.ShapeDtypeStruct((_N, _D), jnp.float32),
        ],
    )(x, wcat, b)


def _mm_mid_body(a0_ref, a1_ref, sp_ref, w_ref, b_ref, h_ref, y_ref, s_ref):
    h = jnp.maximum(a0_ref[...] + a1_ref[...] + sp_ref[...], 0.0)
    h_ref[...] = h
    p = jnp.dot(h, w_ref[...], preferred_element_type=jnp.float32)
    y_ref[...] = p[:, :_BD]
    s_ref[...] = p[:, _BD:] + b_ref[...]


def _mm_mid(a0, a1, sp, wcat, b):
    return pl.pallas_call(
        _mm_mid_body,
        grid=(_N // _RB,),
        in_specs=[
            pl.BlockSpec((_RB, _D), lambda i: (i, 0)),
            pl.BlockSpec((_RB, _D), lambda i: (i, 0)),
            pl.BlockSpec((_RB, _D), lambda i: (i, 0)),
            pl.BlockSpec((_D, _BD + _D), lambda i: (0, 0)),
            pl.BlockSpec((1, _D), lambda i: (0, 0)),
        ],
        out_specs=[
            pl.BlockSpec((_RB, _D), lambda i: (i, 0)),
            pl.BlockSpec((_RB, _BD), lambda i: (i, 0)),
            pl.BlockSpec((_RB, _D), lambda i: (i, 0)),
        ],
        out_shape=[
            jax.ShapeDtypeStruct((_N, _D), jnp.float32),
            jax.ShapeDtypeStruct((_N, _BD), jnp.float32),
            jax.ShapeDtypeStruct((_N, _D), jnp.float32),
        ],
    )(a0, a1, sp, wcat, b)


def _mm_last_body(a0_ref, a1_ref, sp_ref, h_ref):
    h_ref[...] = jnp.maximum(a0_ref[...] + a1_ref[...] + sp_ref[...], 0.0)


def _mm_last(a0, a1, sp):
    return pl.pallas_call(
        _mm_last_body,
        grid=(_N // _RB,),
        in_specs=[
            pl.BlockSpec((_RB, _D), lambda i: (i, 0)),
            pl.BlockSpec((_RB, _D), lambda i: (i, 0)),
            pl.BlockSpec((_RB, _D), lambda i: (i, 0)),
        ],
        out_specs=pl.BlockSpec((_RB, _D), lambda i: (i, 0)),
        out_shape=jax.ShapeDtypeStruct((_N, _D), jnp.float32),
    )(a0, a1, sp)


def _final_body(s1, s2, s3, c, h1, h2, h3, t1, t2, t3, r,
                wg1, wg2, wg3, wh1, wh2, wh3, wt1, wt2, wt3, wr, fb, o_ref):
    cnt = jnp.maximum(c[:_B, :1] + c[_B:, :1], 1.0)
    acc = jnp.sum(((s1[:_B] + s1[_B:]) / cnt) * wg1[...], axis=1,
                  keepdims=True)
    acc = acc + jnp.sum(((s2[:_B] + s2[_B:]) / cnt) * wg2[...], axis=1,
                        keepdims=True)
    acc = acc + jnp.sum(((s3[:_B] + s3[_B:]) / cnt) * wg3[...], axis=1,
                        keepdims=True)
    for hh, ww in ((h1, wh1), (h2, wh2), (h3, wh3),
                   (t1, wt1), (t2, wt2), (t3, wt3), (r, wr)):
        acc = acc + jnp.sum(hh[...] * ww[...], axis=1, keepdims=True)
    o_ref[...] = acc + fb[...]


def _final(*args):
    return pl.pallas_call(
        _final_body,
        out_shape=jax.ShapeDtypeStruct((_B, 1), jnp.float32),
    )(*args)


def _edge_kw():
    return dict(
        out_type=jax.ShapeDtypeStruct((2 * _NPE, _D), jnp.float32),
        mesh=_mesh(),
        compiler_params=pltpu.CompilerParams(needs_layout_passes=False),
        scratch_types=[
            pltpu.VMEM((_ECH,), jnp.int32),
            pltpu.VMEM((_ECH,), jnp.int32),
            pltpu.VMEM((_ECH,), jnp.int32),
            pltpu.VMEM((_NR, _NB), jnp.float32),
            pltpu.VMEM((_ECH, _BD), jnp.float32),
            pltpu.VMEM((_ECH, _D), jnp.float32),
            pltpu.VMEM_SHARED((_NPE, _D), jnp.float32),
            pltpu.SemaphoreType.DMA,
        ],
    )


def _edge_body(y, srcp, dstp, typp, a_l, zrows, out,
               sidx_v, didx_v, typ_v, at_v, g_v, m_v, agg_sh, sem):
    cid = lax.axis_index("c")
    sid = lax.axis_index("s")
    wid = sid * _NC + cid
    rows_slice = pl.ds(sid * _ZR, _ZR)
    pltpu.sync_copy(zrows, agg_sh.at[rows_slice])
    pltpu.sync_copy(a_l, at_v)
    plsc.subcore_barrier()

    lane = jnp.arange(_LN, dtype=jnp.int32)

    def chunk(i, carry):
        base = wid * _EPW + i * _ECH
        pltpu.sync_copy(srcp.at[pl.ds(base, _ECH)], sidx_v)
        pltpu.sync_copy(dstp.at[pl.ds(base, _ECH)], didx_v)
        pltpu.sync_copy(typp.at[pl.ds(base, _ECH)], typ_v)
        pltpu.async_copy(y.at[sidx_v], g_v, sem).wait()
        for j in range(_ECH // _LN):
            typ16 = typ_v[pl.ds(j * _LN, _LN)]
            rows = jnp.full((_LN,), j * _LN, jnp.int32) + lane
            coefs = [
                plsc.load_gather(at_v, [typ16, jnp.full((_LN,), b, jnp.int32)])
                for b in range(_NB)
            ]

            def dbody(d, c2):
                acc = coefs[0] * plsc.load_gather(
                    g_v, [rows, jnp.full((_LN,), 0, jnp.int32) + d])
                for b in range(1, _NB):
                    acc = acc + coefs[b] * plsc.load_gather(
                        g_v, [rows, jnp.full((_LN,), b * _D, jnp.int32) + d])
                plsc.store_scatter(
                    m_v, [rows, jnp.full((_LN,), 0, jnp.int32) + d], acc)
                return c2

            lax.fori_loop(0, _D, dbody, 0)
        pltpu.sync_copy(m_v, agg_sh.at[didx_v], add=True)
        return carry

    lax.fori_loop(0, _ENCH, chunk, 0)
    plsc.subcore_barrier()
    pltpu.sync_copy(agg_sh.at[rows_slice],
                    out.at[pl.ds(cid * _NPE + sid * _ZR, _ZR)])


_edge_kernel = pl.kernel(_edge_body, **_edge_kw())


def _pool_kw():
    return dict(out_type=(
        [jax.ShapeDtypeStruct((2 * _B, _D), jnp.float32)] * _L
        + [jax.ShapeDtypeStruct((2 * _B, _D), jnp.float32)]
        + [jax.ShapeDtypeStruct((_B, _D), jnp.float32)] * _L
        + [jax.ShapeDtypeStruct((_B, _D), jnp.float32)] * _L
        + [jax.ShapeDtypeStruct((_B, _D), jnp.float32)]
    ),
        mesh=_mesh(),
        compiler_params=pltpu.CompilerParams(needs_layout_passes=False),
        scratch_types=[
            pltpu.VMEM((_PCH,), jnp.int32),
            pltpu.VMEM((_PCH, _D), jnp.float32),
            pltpu.VMEM((_PCH, _D), jnp.float32),
            pltpu.VMEM((_BW,), jnp.int32),
            pltpu.VMEM((_BW, _D), jnp.float32),
            pltpu.VMEM_SHARED((_GP, _D), jnp.float32),
            pltpu.VMEM_SHARED((_GP, _D), jnp.float32),
            pltpu.VMEM_SHARED((_GP, _D), jnp.float32),
            pltpu.VMEM_SHARED((_GP, _D), jnp.float32),
            pltpu.SemaphoreType.DMA,
        ],
    )


def _pool_body(h1, h2, h3, gid, head_i, tail_i, rlab_i, remb,
               zsum, zcnt, ones_i,
               s1o, s2o, s3o, co, h1o, h2o, h3o, t1o, t2o, t3o, ro,
               gidx_v, row_v, ones_v, bidx_v, gbuf_v,
               ss1, ss2, ss3, scnt_sh, sem):
    cid = lax.axis_index("c")
    sid = lax.axis_index("s")
    wid = sid * _NC + cid
    zs = pl.ds(sid * _GZR, _GZR)
    pltpu.sync_copy(zsum, ss1.at[zs])
    pltpu.sync_copy(zsum, ss2.at[zs])
    pltpu.sync_copy(zsum, ss3.at[zs])
    pltpu.sync_copy(zcnt, scnt_sh.at[zs])
    pltpu.sync_copy(ones_i, ones_v)
    plsc.subcore_barrier()

    def chunk(i, carry):
        base = wid * _RPW + i * _PCH
        sl = pl.ds(base, _PCH)
        pltpu.sync_copy(gid.at[sl], gidx_v)
        pltpu.sync_copy(h1.at[sl], row_v)
        pltpu.sync_copy(row_v, ss1.at[gidx_v], add=True)
        pltpu.sync_copy(h2.at[sl], row_v)
        pltpu.sync_copy(row_v, ss2.at[gidx_v], add=True)
        pltpu.sync_copy(h3.at[sl], row_v)
        pltpu.sync_copy(row_v, ss3.at[gidx_v], add=True)
        pltpu.sync_copy(ones_v, scnt_sh.at[gidx_v], add=True)
        return carry

    lax.fori_loop(0, _PNCH, chunk, 0)
    plsc.subcore_barrier()

    dump = pl.ds(sid * (_B // _NS), _B // _NS)
    dump_o = pl.ds(cid * _B + sid * (_B // _NS), _B // _NS)
    pltpu.sync_copy(ss1.at[dump], s1o.at[dump_o])
    pltpu.sync_copy(ss2.at[dump], s2o.at[dump_o])
    pltpu.sync_copy(ss3.at[dump], s3o.at[dump_o])
    pltpu.sync_copy(scnt_sh.at[dump], co.at[dump_o])

    b0 = pl.ds(wid * _BW, _BW)
    pltpu.sync_copy(head_i.at[b0], bidx_v)
    pltpu.async_copy(h1.at[bidx_v], gbuf_v, sem).wait()
    pltpu.sync_copy(gbuf_v, h1o.at[b0])
    pltpu.async_copy(h2.at[bidx_v], gbuf_v, sem).wait()
    pltpu.sync_copy(gbuf_v, h2o.at[b0])
    pltpu.async_copy(h3.at[bidx_v], gbuf_v, sem).wait()
    pltpu.sync_copy(gbuf_v, h3o.at[b0])
    pltpu.sync_copy(tail_i.at[b0], bidx_v)
    pltpu.async_copy(h1.at[bidx_v], gbuf_v, sem).wait()
    pltpu.sync_copy(gbuf_v, t1o.at[b0])
    pltpu.async_copy(h2.at[bidx_v], gbuf_v, sem).wait()
    pltpu.sync_copy(gbuf_v, t2o.at[b0])
    pltpu.async_copy(h3.at[bidx_v], gbuf_v, sem).wait()
    pltpu.sync_copy(gbuf_v, t3o.at[b0])
    pltpu.sync_copy(rlab_i.at[b0], bidx_v)
    pltpu.async_copy(remb.at[bidx_v], gbuf_v, sem).wait()
    pltpu.sync_copy(gbuf_v, ro.at[b0])


_pool_kernel = pl.kernel(_pool_body, **_pool_kw())


def kernel(x, edge_index, edge_type, graph_ids, head_ids, tail_ids,
           rel_labels, V, A, W_self, bias, rel_emb, fcW, fcb):
    f32 = jnp.float32
    i32 = jnp.int32
    src = edge_index[0].astype(i32)
    dst = edge_index[1].astype(i32)
    typ = edge_type.astype(i32)
    pad = _EP - _E
    srcp = jnp.concatenate([src, jnp.zeros((pad,), i32)])
    dstp = jnp.concatenate([dst, jnp.full((pad,), _N, i32)])
    typp = jnp.concatenate([typ, jnp.zeros((pad,), i32)])
    wcat = [jnp.concatenate(
        [V[l].transpose(1, 0, 2).reshape(_D, _BD), W_self[l]], axis=1)
        for l in range(_L)]
    brow = [bias[l].reshape(1, _D) for l in range(_L)]
    zrows = jnp.zeros((_ZR, _D), f32)

    y0, s0 = _mm_first(x, wcat[0], brow[0])
    ag = _edge_kernel(y0, srcp, dstp, typp, A[0], zrows)
    h1, y1, s1 = _mm_mid(ag[:_N], ag[_NPE:_NPE + _N], s0, wcat[1], brow[1])
    ag = _edge_kernel(y1, srcp, dstp, typp, A[1], zrows)
    h2, y2, s2 = _mm_mid(ag[:_N], ag[_NPE:_NPE + _N], s1, wcat[2], brow[2])
    ag = _edge_kernel(y2, srcp, dstp, typp, A[2], zrows)
    h3 = _mm_last(ag[:_N], ag[_NPE:_NPE + _N], s2)

    npad = jnp.zeros((_NP2 - _N, _D), f32)
    h1p = jnp.concatenate([h1, npad], axis=0)
    h2p = jnp.concatenate([h2, npad], axis=0)
    h3p = jnp.concatenate([h3, npad], axis=0)
    gidp = jnp.concatenate(
        [graph_ids.astype(i32), jnp.full((_NP2 - _N,), _B, i32)])
    zsum = jnp.zeros((_GZR, _D), f32)
    zcnt = jnp.zeros((_GZR, _D), f32)
    ones_i = jnp.ones((_PCH, _D), f32)

    pooled = _pool_kernel(
        h1p, h2p, h3p, gidp, head_ids.astype(i32), tail_ids.astype(i32),
        rel_labels.astype(i32), rel_emb, zsum, zcnt, ones_i)

    ws = [fcW[k * _D:(k + 1) * _D, 0].reshape(1, _D) for k in range(10)]
    fb = fcb.reshape(1, 1)
    return _final(*pooled, *ws, fb)


# speedup vs baseline: 1.0513x; 1.0513x over previous
import functools

import jax
import jax.numpy as jnp
from jax import lax
from jax.experimental import pallas as pl
from jax.experimental.pallas import tpu as pltpu
from jax.experimental.pallas import tpu_sc as plsc

_N, _D, _L, _NB, _NR, _B, _E = 10000, 128, 3, 4, 64, 512, 160000
_BD = _NB * _D

_NC, _NS, _LN = 2, 16, 16
_NW = _NC * _NS

_EP = 163840
_EPW = _EP // _NW
_ECH = 32
_ENCH = _EPW // _ECH
_NPE = 10112
_ZR = _NPE // _NS

_NP2 = 10240
_RPW = _NP2 // _NW
_PCH = 64
_PNCH = _RPW // _PCH
_GP = 640
_GZR = _GP // _NS
_BW = _B // _NW

_RB = 400


def _mesh():
    return plsc.VectorSubcoreMesh(core_axis_name="c", subcore_axis_name="s")


def _mm_first_body(x_ref, w_ref, b_ref, y_ref, s_ref):
    p = jnp.dot(x_ref[...], w_ref[...], preferred_element_type=jnp.float32)
    y_ref[...] = p[:, :_BD]
    s_ref[...] = p[:, _BD:] + b_ref[...]


def _mm_first(x, wcat, b):
    return pl.pallas_call(
        _mm_first_body,
        grid=(_N // _RB,),
        in_specs=[
            pl.BlockSpec((_RB, _D), lambda i: (i, 0)),
            pl.BlockSpec((_D, _BD + _D), lambda i: (0, 0)),
            pl.BlockSpec((1, _D), lambda i: (0, 0)),
        ],
        out_specs=[
            pl.BlockSpec((_RB, _BD), lambda i: (i, 0)),
            pl.BlockSpec((_RB, _D), lambda i: (i, 0)),
        ],
        out_shape=[
            jax.ShapeDtypeStruct((_N, _BD), jnp.float32),
            jax.ShapeDtypeStruct((_N, _D), jnp.float32),
        ],
    )(x, wcat, b)


def _mm_mid_body(a0_ref, a1_ref, sp_ref, w_ref, b_ref, h_ref, y_ref, s_ref):
    h = jnp.maximum(a0_ref[...] + a1_ref[...] + sp_ref[...], 0.0)
    h_ref[...] = h
    p = jnp.dot(h, w_ref[...], preferred_element_type=jnp.float32)
    y_ref[...] = p[:, :_BD]
    s_ref[...] = p[:, _BD:] + b_ref[...]


def _mm_mid(a0, a1, sp, wcat, b):
    return pl.pallas_call(
        _mm_mid_body,
        grid=(_N // _RB,),
        in_specs=[
            pl.BlockSpec((_RB, _D), lambda i: (i, 0)),
            pl.BlockSpec((_RB, _D), lambda i: (i, 0)),
            pl.BlockSpec((_RB, _D), lambda i: (i, 0)),
            pl.BlockSpec((_D, _BD + _D), lambda i: (0, 0)),
            pl.BlockSpec((1, _D), lambda i: (0, 0)),
        ],
        out_specs=[
            pl.BlockSpec((_RB, _D), lambda i: (i, 0)),
            pl.BlockSpec((_RB, _BD), lambda i: (i, 0)),
            pl.BlockSpec((_RB, _D), lambda i: (i, 0)),
        ],
        out_shape=[
            jax.ShapeDtypeStruct((_N, _D), jnp.float32),
            jax.ShapeDtypeStruct((_N, _BD), jnp.float32),
            jax.ShapeDtypeStruct((_N, _D), jnp.float32),
        ],
    )(a0, a1, sp, wcat, b)


def _mm_last_body(a0_ref, a1_ref, sp_ref, h_ref):
    h_ref[...] = jnp.maximum(a0_ref[...] + a1_ref[...] + sp_ref[...], 0.0)


def _mm_last(a0, a1, sp):
    return pl.pallas_call(
        _mm_last_body,
        grid=(_N // _RB,),
        in_specs=[
            pl.BlockSpec((_RB, _D), lambda i: (i, 0)),
            pl.BlockSpec((_RB, _D), lambda i: (i, 0)),
            pl.BlockSpec((_RB, _D), lambda i: (i, 0)),
        ],
        out_specs=pl.BlockSpec((_RB, _D), lambda i: (i, 0)),
        out_shape=jax.ShapeDtypeStruct((_N, _D), jnp.float32),
    )(a0, a1, sp)


def _final_body(s1, s2, s3, c, h1, h2, h3, t1, t2, t3, r,
                wg1, wg2, wg3, wh1, wh2, wh3, wt1, wt2, wt3, wr, fb, o_ref):
    cnt = jnp.maximum(c[:_B, :1] + c[_B:, :1], 1.0)
    acc = jnp.sum(((s1[:_B] + s1[_B:]) / cnt) * wg1[...], axis=1,
                  keepdims=True)
    acc = acc + jnp.sum(((s2[:_B] + s2[_B:]) / cnt) * wg2[...], axis=1,
                        keepdims=True)
    acc = acc + jnp.sum(((s3[:_B] + s3[_B:]) / cnt) * wg3[...], axis=1,
                        keepdims=True)
    for hh, ww in ((h1, wh1), (h2, wh2), (h3, wh3),
                   (t1, wt1), (t2, wt2), (t3, wt3), (r, wr)):
        acc = acc + jnp.sum(hh[...] * ww[...], axis=1, keepdims=True)
    o_ref[...] = acc + fb[...]


def _final(*args):
    return pl.pallas_call(
        _final_body,
        out_shape=jax.ShapeDtypeStruct((_B, 1), jnp.float32),
    )(*args)


def _edge_kw():
    return dict(
        out_type=jax.ShapeDtypeStruct((2 * _NPE, _D), jnp.float32),
        mesh=_mesh(),
        compiler_params=pltpu.CompilerParams(needs_layout_passes=False),
        scratch_types=[
            pltpu.VMEM((_ECH,), jnp.int32),
            pltpu.VMEM((_ECH,), jnp.int32),
            pltpu.VMEM((_ECH,), jnp.int32),
            pltpu.VMEM((_ECH,), jnp.int32),
            pltpu.VMEM((_ECH,), jnp.int32),
            pltpu.VMEM((_ECH,), jnp.int32),
            pltpu.VMEM((_NR, _NB), jnp.float32),
            pltpu.VMEM((_ECH, _BD), jnp.float32),
            pltpu.VMEM((_ECH, _BD), jnp.float32),
            pltpu.VMEM((_ECH, _D), jnp.float32),
            pltpu.VMEM_SHARED((_NPE, _D), jnp.float32),
            pltpu.SemaphoreType.DMA,
            pltpu.SemaphoreType.DMA,
        ],
    )


def _edge_body(y, srcp, dstp, typp, a_l, zrows, out,
               s0v, d0v, t0v, s1v, d1v, t1v,
               at_v, g0, g1, m0, agg_sh, sem0, sem1):
    cid = lax.axis_index("c")
    sid = lax.axis_index("s")
    wid = sid * _NC + cid
    rows_slice = pl.ds(sid * _ZR, _ZR)
    pltpu.sync_copy(zrows, agg_sh.at[rows_slice])
    pltpu.sync_copy(a_l, at_v)
    plsc.subcore_barrier()

    lane = jnp.arange(_LN, dtype=jnp.int32)
    wbase = wid * _EPW

    def stage(c, sv, dv, tv):
        base = wbase + c * _ECH
        pltpu.sync_copy(srcp.at[pl.ds(base, _ECH)], sv)
        pltpu.sync_copy(dstp.at[pl.ds(base, _ECH)], dv)
        pltpu.sync_copy(typp.at[pl.ds(base, _ECH)], tv)

    def compute(tv, dv, g, m):
        for j in range(_ECH // _LN):
            typ16 = tv[pl.ds(j * _LN, _LN)]
            rows = jnp.full((_LN,), j * _LN, jnp.int32) + lane
            coefs = [
                plsc.load_gather(at_v, [typ16, jnp.full((_LN,), b, jnp.int32)])
                for b in range(_NB)
            ]
            cbase = [jnp.full((_LN,), b * _D, jnp.int32) for b in range(_NB)]

            def qbody(q, c2):
                d0 = q * 4
                for k in range(4):
                    dd = d0 + k
                    i0 = cbase[0] + dd
                    acc = coefs[0] * plsc.load_gather(g, [rows, i0])
                    for b in range(1, _NB):
                        acc = acc + coefs[b] * plsc.load_gather(
                            g, [rows, cbase[b] + dd])
                    plsc.store_scatter(m, [rows, i0], acc)
                return c2

            lax.fori_loop(0, _D // 4, qbody, 0)
        pltpu.sync_copy(m, agg_sh.at[dv], add=True)

    stage(0, s0v, d0v, t0v)
    pltpu.async_copy(y.at[s0v], g0, sem0)

    def step(s, carry):
        c0 = 2 * s
        stage(c0 + 1, s1v, d1v, t1v)
        pltpu.async_copy(y.at[s1v], g1, sem1)
        pltpu.make_async_copy(y.at[s0v], g0, sem0).wait()
        compute(t0v, d0v, g0, m0)
        stage(c0 + 2, s0v, d0v, t0v)
        pltpu.async_copy(y.at[s0v], g0, sem0)
        pltpu.make_async_copy(y.at[s1v], g1, sem1).wait()
        compute(t1v, d1v, g1, m0)
        return carry

    lax.fori_loop(0, _ENCH // 2, step, 0)
    pltpu.make_async_copy(y.at[s0v], g0, sem0).wait()

    plsc.subcore_barrier()
    pltpu.sync_copy(agg_sh.at[rows_slice],
                    out.at[pl.ds(cid * _NPE + sid * _ZR, _ZR)])


_edge_kernel = pl.kernel(_edge_body, **_edge_kw())


def _pool_kw():
    return dict(out_type=(
        [jax.ShapeDtypeStruct((2 * _B, _D), jnp.float32)] * _L
        + [jax.ShapeDtypeStruct((2 * _B, _D), jnp.float32)]
        + [jax.ShapeDtypeStruct((_B, _D), jnp.float32)] * _L
        + [jax.ShapeDtypeStruct((_B, _D), jnp.float32)] * _L
        + [jax.ShapeDtypeStruct((_B, _D), jnp.float32)]
    ),
        mesh=_mesh(),
        compiler_params=pltpu.CompilerParams(needs_layout_passes=False),
        scratch_types=[
            pltpu.VMEM((_PCH,), jnp.int32),
            pltpu.VMEM((_PCH, _D), jnp.float32),
            pltpu.VMEM((_PCH, _D), jnp.float32),
            pltpu.VMEM((_BW,), jnp.int32),
            pltpu.VMEM((_BW, _D), jnp.float32),
            pltpu.VMEM_SHARED((_GP, _D), jnp.float32),
            pltpu.VMEM_SHARED((_GP, _D), jnp.float32),
            pltpu.VMEM_SHARED((_GP, _D), jnp.float32),
            pltpu.VMEM_SHARED((_GP, _D), jnp.float32),
            pltpu.SemaphoreType.DMA,
        ],
    )


def _pool_body(h1, h2, h3, gid, head_i, tail_i, rlab_i, remb,
               zsum, zcnt, ones_i,
               s1o, s2o, s3o, co, h1o, h2o, h3o, t1o, t2o, t3o, ro,
               gidx_v, row_v, ones_v, bidx_v, gbuf_v,
               ss1, ss2, ss3, scnt_sh, sem):
    cid = lax.axis_index("c")
    sid = lax.axis_index("s")
    wid = sid * _NC + cid
    zs = pl.ds(sid * _GZR, _GZR)
    pltpu.sync_copy(zsum, ss1.at[zs])
    pltpu.sync_copy(zsum, ss2.at[zs])
    pltpu.sync_copy(zsum, ss3.at[zs])
    pltpu.sync_copy(zcnt, scnt_sh.at[zs])
    pltpu.sync_copy(ones_i, ones_v)
    plsc.subcore_barrier()

    def chunk(i, carry):
        base = wid * _RPW + i * _PCH
        sl = pl.ds(base, _PCH)
        pltpu.sync_copy(gid.at[sl], gidx_v)
        pltpu.sync_copy(h1.at[sl], row_v)
        pltpu.sync_copy(row_v, ss1.at[gidx_v], add=True)
        pltpu.sync_copy(h2.at[sl], row_v)
        pltpu.sync_copy(row_v, ss2.at[gidx_v], add=True)
        pltpu.sync_copy(h3.at[sl], row_v)
        pltpu.sync_copy(row_v, ss3.at[gidx_v], add=True)
        pltpu.sync_copy(ones_v, scnt_sh.at[gidx_v], add=True)
        return carry

    lax.fori_loop(0, _PNCH, chunk, 0)
    plsc.subcore_barrier()

    dump = pl.ds(sid * (_B // _NS), _B // _NS)
    dump_o = pl.ds(cid * _B + sid * (_B // _NS), _B // _NS)
    pltpu.sync_copy(ss1.at[dump], s1o.at[dump_o])
    pltpu.sync_copy(ss2.at[dump], s2o.at[dump_o])
    pltpu.sync_copy(ss3.at[dump], s3o.at[dump_o])
    pltpu.sync_copy(scnt_sh.at[dump], co.at[dump_o])

    b0 = pl.ds(wid * _BW, _BW)
    pltpu.sync_copy(head_i.at[b0], bidx_v)
    pltpu.async_copy(h1.at[bidx_v], gbuf_v, sem).wait()
    pltpu.sync_copy(gbuf_v, h1o.at[b0])
    pltpu.async_copy(h2.at[bidx_v], gbuf_v, sem).wait()
    pltpu.sync_copy(gbuf_v, h2o.at[b0])
    pltpu.async_copy(h3.at[bidx_v], gbuf_v, sem).wait()
    pltpu.sync_copy(gbuf_v, h3o.at[b0])
    pltpu.sync_copy(tail_i.at[b0], bidx_v)
    pltpu.async_copy(h1.at[bidx_v], gbuf_v, sem).wait()
    pltpu.sync_copy(gbuf_v, t1o.at[b0])
    pltpu.async_copy(h2.at[bidx_v], gbuf_v, sem).wait()
    pltpu.sync_copy(gbuf_v, t2o.at[b0])
    pltpu.async_copy(h3.at[bidx_v], gbuf_v, sem).wait()
    pltpu.sync_copy(gbuf_v, t3o.at[b0])
    pltpu.sync_copy(rlab_i.at[b0], bidx_v)
    pltpu.async_copy(remb.at[bidx_v], gbuf_v, sem).wait()
    pltpu.sync_copy(gbuf_v, ro.at[b0])


_pool_kernel = pl.kernel(_pool_body, **_pool_kw())


def kernel(x, edge_index, edge_type, graph_ids, head_ids, tail_ids,
           rel_labels, V, A, W_self, bias, rel_emb, fcW, fcb):
    f32 = jnp.float32
    i32 = jnp.int32
    src = edge_index[0].astype(i32)
    dst = edge_index[1].astype(i32)
    typ = edge_type.astype(i32)
    pad = _EP + _ECH - _E
    srcp = jnp.concatenate([src, jnp.zeros((pad,), i32)])
    dstp = jnp.concatenate([dst, jnp.full((pad,), _N, i32)])
    typp = jnp.concatenate([typ, jnp.zeros((pad,), i32)])
    wcat = [jnp.concatenate(
        [V[l].transpose(1, 0, 2).reshape(_D, _BD), W_self[l]], axis=1)
        for l in range(_L)]
    brow = [bias[l].reshape(1, _D) for l in range(_L)]
    zrows = jnp.zeros((_ZR, _D), f32)

    y0, s0 = _mm_first(x, wcat[0], brow[0])
    ag = _edge_kernel(y0, srcp, dstp, typp, A[0], zrows)
    h1, y1, s1 = _mm_mid(ag[:_N], ag[_NPE:_NPE + _N], s0, wcat[1], brow[1])
    ag = _edge_kernel(y1, srcp, dstp, typp, A[1], zrows)
    h2, y2, s2 = _mm_mid(ag[:_N], ag[_NPE:_NPE + _N], s1, wcat[2], brow[2])
    ag = _edge_kernel(y2, srcp, dstp, typp, A[2], zrows)
    h3 = _mm_last(ag[:_N], ag[_NPE:_NPE + _N], s2)

    npad = jnp.zeros((_NP2 - _N, _D), f32)
    h1p = jnp.concatenate([h1, npad], axis=0)
    h2p = jnp.concatenate([h2, npad], axis=0)
    h3p = jnp.concatenate([h3, npad], axis=0)
    gidp = jnp.concatenate(
        [graph_ids.astype(i32), jnp.full((_NP2 - _N,), _B, i32)])
    zsum = jnp.zeros((_GZR, _D), f32)
    zcnt = jnp.zeros((_GZR, _D), f32)
    ones_i = jnp.ones((_PCH, _D), f32)

    pooled = _pool_kernel(
        h1p, h2p, h3p, gidp, head_ids.astype(i32), tail_ids.astype(i32),
        rel_labels.astype(i32), rel_emb, zsum, zcnt, ones_i)

    ws = [fcW[k * _D:(k + 1) * _D, 0].reshape(1, _D) for k in range(10)]
    fb = fcb.reshape(1, 1)
    return _final(*pooled, *ws, fb)


# speedup vs baseline: 1.5369x; 1.4619x over previous
import functools

import jax
import jax.numpy as jnp
from jax import lax
from jax.experimental import pallas as pl
from jax.experimental.pallas import tpu as pltpu
from jax.experimental.pallas import tpu_sc as plsc

_N, _D, _L, _NB, _NR, _B, _E = 10000, 128, 3, 4, 64, 512, 160000
_BD = _NB * _D
_BDP = _BD // 2

_NC, _NS, _LN = 2, 16, 16
_NW = _NC * _NS

_EP = 163840
_EPW = _EP // _NW
_ECH = 32
_ENCH = _EPW // _ECH
_NPE = 10112
_ZR = _NPE // _NS

_NP2 = 10240
_RPW = _NP2 // _NW
_PCH = 64
_PNCH = _RPW // _PCH
_GP = 640
_GZR = _GP // _NS
_BW = _B // _NW

_RB = 400


def _mesh():
    return plsc.VectorSubcoreMesh(core_axis_name="c", subcore_axis_name="s")


def _pack_y(p):
    u = lax.bitcast_convert_type(p.astype(jnp.bfloat16), jnp.uint16)
    u = u.astype(jnp.uint32)
    p01 = u[:, 0 * _D:1 * _D] | (u[:, 1 * _D:2 * _D] << 16)
    p23 = u[:, 2 * _D:3 * _D] | (u[:, 3 * _D:4 * _D] << 16)
    return lax.bitcast_convert_type(
        jnp.concatenate([p01, p23], axis=1), jnp.int32)


def _mm_first_body(x_ref, w_ref, b_ref, y_ref, s_ref):
    p = jnp.dot(x_ref[...], w_ref[...], preferred_element_type=jnp.float32)
    y_ref[...] = _pack_y(p[:, :_BD])
    s_ref[...] = p[:, _BD:] + b_ref[...]


def _mm_first(x, wcat, b):
    return pl.pallas_call(
        _mm_first_body,
        grid=(_N // _RB,),
        in_specs=[
            pl.BlockSpec((_RB, _D), lambda i: (i, 0)),
            pl.BlockSpec((_D, _BD + _D), lambda i: (0, 0)),
            pl.BlockSpec((1, _D), lambda i: (0, 0)),
        ],
        out_specs=[
            pl.BlockSpec((_RB, _BDP), lambda i: (i, 0)),
            pl.BlockSpec((_RB, _D), lambda i: (i, 0)),
        ],
        out_shape=[
            jax.ShapeDtypeStruct((_N, _BDP), jnp.int32),
            jax.ShapeDtypeStruct((_N, _D), jnp.float32),
        ],
    )(x, wcat, b)


def _mm_mid_body(a0_ref, a1_ref, sp_ref, w_ref, b_ref, h_ref, y_ref, s_ref):
    h = jnp.maximum(a0_ref[...] + a1_ref[...] + sp_ref[...], 0.0)
    h_ref[...] = h
    p = jnp.dot(h, w_ref[...], preferred_element_type=jnp.float32)
    y_ref[...] = _pack_y(p[:, :_BD])
    s_ref[...] = p[:, _BD:] + b_ref[...]


def _mm_mid(a0, a1, sp, wcat, b):
    return pl.pallas_call(
        _mm_mid_body,
        grid=(_N // _RB,),
        in_specs=[
            pl.BlockSpec((_RB, _D), lambda i: (i, 0)),
            pl.BlockSpec((_RB, _D), lambda i: (i, 0)),
            pl.BlockSpec((_RB, _D), lambda i: (i, 0)),
            pl.BlockSpec((_D, _BD + _D), lambda i: (0, 0)),
            pl.BlockSpec((1, _D), lambda i: (0, 0)),
        ],
        out_specs=[
            pl.BlockSpec((_RB, _D), lambda i: (i, 0)),
            pl.BlockSpec((_RB, _BDP), lambda i: (i, 0)),
            pl.BlockSpec((_RB, _D), lambda i: (i, 0)),
        ],
        out_shape=[
            jax.ShapeDtypeStruct((_N, _D), jnp.float32),
            jax.ShapeDtypeStruct((_N, _BDP), jnp.int32),
            jax.ShapeDtypeStruct((_N, _D), jnp.float32),
        ],
    )(a0, a1, sp, wcat, b)


def _mm_last_body(a0_ref, a1_ref, sp_ref, h_ref):
    h_ref[...] = jnp.maximum(a0_ref[...] + a1_ref[...] + sp_ref[...], 0.0)


def _mm_last(a0, a1, sp):
    return pl.pallas_call(
        _mm_last_body,
        grid=(_N // _RB,),
        in_specs=[
            pl.BlockSpec((_RB, _D), lambda i: (i, 0)),
            pl.BlockSpec((_RB, _D), lambda i: (i, 0)),
            pl.BlockSpec((_RB, _D), lambda i: (i, 0)),
        ],
        out_specs=pl.BlockSpec((_RB, _D), lambda i: (i, 0)),
        out_shape=jax.ShapeDtypeStruct((_N, _D), jnp.float32),
    )(a0, a1, sp)


def _final_body(s1, s2, s3, c, h1, h2, h3, t1, t2, t3, r,
                wg1, wg2, wg3, wh1, wh2, wh3, wt1, wt2, wt3, wr, fb, o_ref):
    cnt = jnp.maximum(c[:_B, :1] + c[_B:, :1], 1.0)
    acc = jnp.sum(((s1[:_B] + s1[_B:]) / cnt) * wg1[...], axis=1,
                  keepdims=True)
    acc = acc + jnp.sum(((s2[:_B] + s2[_B:]) / cnt) * wg2[...], axis=1,
                        keepdims=True)
    acc = acc + jnp.sum(((s3[:_B] + s3[_B:]) / cnt) * wg3[...], axis=1,
                        keepdims=True)
    for hh, ww in ((h1, wh1), (h2, wh2), (h3, wh3),
                   (t1, wt1), (t2, wt2), (t3, wt3), (r, wr)):
        acc = acc + jnp.sum(hh[...] * ww[...], axis=1, keepdims=True)
    o_ref[...] = acc + fb[...]


def _final(*args):
    return pl.pallas_call(
        _final_body,
        out_shape=jax.ShapeDtypeStruct((_B, 1), jnp.float32),
    )(*args)


def _edge_kw():
    return dict(
        out_type=jax.ShapeDtypeStruct((2 * _NPE, _D), jnp.float32),
        mesh=_mesh(),
        compiler_params=pltpu.CompilerParams(needs_layout_passes=False),
        scratch_types=[
            pltpu.VMEM((_ECH,), jnp.int32),
            pltpu.VMEM((_ECH,), jnp.int32),
            pltpu.VMEM((_ECH,), jnp.int32),
            pltpu.VMEM((_ECH,), jnp.int32),
            pltpu.VMEM((_ECH,), jnp.int32),
            pltpu.VMEM((_ECH,), jnp.int32),
            pltpu.VMEM((_NR, _NB), jnp.float32),
            pltpu.VMEM((_ECH, _BDP), jnp.int32),
            pltpu.VMEM((_ECH, _BDP), jnp.int32),
            pltpu.VMEM((_ECH, _D), jnp.float32),
            pltpu.VMEM_SHARED((_NPE, _D), jnp.float32),
            pltpu.SemaphoreType.DMA,
            pltpu.SemaphoreType.DMA,
        ],
    )


def _edge_body(y, srcp, dstp, typp, a_l, zrows, out,
               s0v, d0v, t0v, s1v, d1v, t1v,
               at_v, g0, g1, m0, agg_sh, sem0, sem1):
    cid = lax.axis_index("c")
    sid = lax.axis_index("s")
    wid = sid * _NC + cid
    rows_slice = pl.ds(sid * _ZR, _ZR)
    pltpu.sync_copy(zrows, agg_sh.at[rows_slice])
    pltpu.sync_copy(a_l, at_v)
    plsc.subcore_barrier()

    lane = jnp.arange(_LN, dtype=jnp.int32)
    wbase = wid * _EPW

    def stage(c, sv, dv, tv):
        base = wbase + c * _ECH
        pltpu.sync_copy(srcp.at[pl.ds(base, _ECH)], sv)
        pltpu.sync_copy(dstp.at[pl.ds(base, _ECH)], dv)
        pltpu.sync_copy(typp.at[pl.ds(base, _ECH)], tv)

    def compute(tv, dv, g, m):
        for j in range(_ECH // _LN):
            typ16 = tv[pl.ds(j * _LN, _LN)]
            rows = jnp.full((_LN,), j * _LN, jnp.int32) + lane
            coefs = [
                plsc.load_gather(at_v, [typ16, jnp.full((_LN,), b, jnp.int32)])
                for b in range(_NB)
            ]
            cb1 = jnp.full((_LN,), _D, jnp.int32)
            mask = jnp.full((_LN,), -65536, jnp.int32)

            def unpack(v):
                lo = lax.bitcast_convert_type(v << 16, jnp.float32)
                hi = lax.bitcast_convert_type(v & mask, jnp.float32)
                return lo, hi

            def qbody(q, c2):
                d0 = q * 4
                for k in range(4):
                    dd = d0 + k
                    i0 = jnp.full((_LN,), dd, jnp.int32)
                    f0, f1 = unpack(plsc.load_gather(g, [rows, i0]))
                    f2, f3 = unpack(plsc.load_gather(g, [rows, cb1 + dd]))
                    acc = (coefs[0] * f0 + coefs[1] * f1
                           + coefs[2] * f2 + coefs[3] * f3)
                    plsc.store_scatter(m, [rows, i0], acc)
                return c2

            lax.fori_loop(0, _D // 4, qbody, 0)
        pltpu.sync_copy(m, agg_sh.at[dv], add=True)

    stage(0, s0v, d0v, t0v)
    pltpu.async_copy(y.at[s0v], g0, sem0)

    def step(s, carry):
        c0 = 2 * s
        stage(c0 + 1, s1v, d1v, t1v)
        pltpu.async_copy(y.at[s1v], g1, sem1)
        pltpu.make_async_copy(y.at[s0v], g0, sem0).wait()
        compute(t0v, d0v, g0, m0)
        stage(c0 + 2, s0v, d0v, t0v)
        pltpu.async_copy(y.at[s0v], g0, sem0)
        pltpu.make_async_copy(y.at[s1v], g1, sem1).wait()
        compute(t1v, d1v, g1, m0)
        return carry

    lax.fori_loop(0, _ENCH // 2, step, 0)
    pltpu.make_async_copy(y.at[s0v], g0, sem0).wait()

    plsc.subcore_barrier()
    pltpu.sync_copy(agg_sh.at[rows_slice],
                    out.at[pl.ds(cid * _NPE + sid * _ZR, _ZR)])


_edge_kernel = pl.kernel(_edge_body, **_edge_kw())


def _pool_kw():
    return dict(out_type=(
        [jax.ShapeDtypeStruct((2 * _B, _D), jnp.float32)] * _L
        + [jax.ShapeDtypeStruct((2 * _B, _D), jnp.float32)]
        + [jax.ShapeDtypeStruct((_B, _D), jnp.float32)] * _L
        + [jax.ShapeDtypeStruct((_B, _D), jnp.float32)] * _L
        + [jax.ShapeDtypeStruct((_B, _D), jnp.float32)]
    ),
        mesh=_mesh(),
        compiler_params=pltpu.CompilerParams(needs_layout_passes=False),
        scratch_types=[
            pltpu.VMEM((_PCH,), jnp.int32),
            pltpu.VMEM((_PCH, _D), jnp.float32),
            pltpu.VMEM((_PCH, _D), jnp.float32),
            pltpu.VMEM((_BW,), jnp.int32),
            pltpu.VMEM((_BW, _D), jnp.float32),
            pltpu.VMEM_SHARED((_GP, _D), jnp.float32),
            pltpu.VMEM_SHARED((_GP, _D), jnp.float32),
            pltpu.VMEM_SHARED((_GP, _D), jnp.float32),
            pltpu.VMEM_SHARED((_GP, _D), jnp.float32),
            pltpu.SemaphoreType.DMA,
        ],
    )


def _pool_body(h1, h2, h3, gid, head_i, tail_i, rlab_i, remb,
               zsum, zcnt, ones_i,
               s1o, s2o, s3o, co, h1o, h2o, h3o, t1o, t2o, t3o, ro,
               gidx_v, row_v, ones_v, bidx_v, gbuf_v,
               ss1, ss2, ss3, scnt_sh, sem):
    cid = lax.axis_index("c")
    sid = lax.axis_index("s")
    wid = sid * _NC + cid
    zs = pl.ds(sid * _GZR, _GZR)
    pltpu.sync_copy(zsum, ss1.at[zs])
    pltpu.sync_copy(zsum, ss2.at[zs])
    pltpu.sync_copy(zsum, ss3.at[zs])
    pltpu.sync_copy(zcnt, scnt_sh.at[zs])
    pltpu.sync_copy(ones_i, ones_v)
    plsc.subcore_barrier()

    def chunk(i, carry):
        base = wid * _RPW + i * _PCH
        sl = pl.ds(base, _PCH)
        pltpu.sync_copy(gid.at[sl], gidx_v)
        pltpu.sync_copy(h1.at[sl], row_v)
        pltpu.sync_copy(row_v, ss1.at[gidx_v], add=True)
        pltpu.sync_copy(h2.at[sl], row_v)
        pltpu.sync_copy(row_v, ss2.at[gidx_v], add=True)
        pltpu.sync_copy(h3.at[sl], row_v)
        pltpu.sync_copy(row_v, ss3.at[gidx_v], add=True)
        pltpu.sync_copy(ones_v, scnt_sh.at[gidx_v], add=True)
        return carry

    lax.fori_loop(0, _PNCH, chunk, 0)
    plsc.subcore_barrier()

    dump = pl.ds(sid * (_B // _NS), _B // _NS)
    dump_o = pl.ds(cid * _B + sid * (_B // _NS), _B // _NS)
    pltpu.sync_copy(ss1.at[dump], s1o.at[dump_o])
    pltpu.sync_copy(ss2.at[dump], s2o.at[dump_o])
    pltpu.sync_copy(ss3.at[dump], s3o.at[dump_o])
    pltpu.sync_copy(scnt_sh.at[dump], co.at[dump_o])

    b0 = pl.ds(wid * _BW, _BW)
    pltpu.sync_copy(head_i.at[b0], bidx_v)
    pltpu.async_copy(h1.at[bidx_v], gbuf_v, sem).wait()
    pltpu.sync_copy(gbuf_v, h1o.at[b0])
    pltpu.async_copy(h2.at[bidx_v], gbuf_v, sem).wait()
    pltpu.sync_copy(gbuf_v, h2o.at[b0])
    pltpu.async_copy(h3.at[bidx_v], gbuf_v, sem).wait()
    pltpu.sync_copy(gbuf_v, h3o.at[b0])
    pltpu.sync_copy(tail_i.at[b0], bidx_v)
    pltpu.async_copy(h1.at[bidx_v], gbuf_v, sem).wait()
    pltpu.sync_copy(gbuf_v, t1o.at[b0])
    pltpu.async_copy(h2.at[bidx_v], gbuf_v, sem).wait()
    pltpu.sync_copy(gbuf_v, t2o.at[b0])
    pltpu.async_copy(h3.at[bidx_v], gbuf_v, sem).wait()
    pltpu.sync_copy(gbuf_v, t3o.at[b0])
    pltpu.sync_copy(rlab_i.at[b0], bidx_v)
    pltpu.async_copy(remb.at[bidx_v], gbuf_v, sem).wait()
    pltpu.sync_copy(gbuf_v, ro.at[b0])


_pool_kernel = pl.kernel(_pool_body, **_pool_kw())


def kernel(x, edge_index, edge_type, graph_ids, head_ids, tail_ids,
           rel_labels, V, A, W_self, bias, rel_emb, fcW, fcb):
    f32 = jnp.float32
    i32 = jnp.int32
    src = edge_index[0].astype(i32)
    dst = edge_index[1].astype(i32)
    typ = edge_type.astype(i32)
    pad = _EP + _ECH - _E
    srcp = jnp.concatenate([src, jnp.zeros((pad,), i32)])
    dstp = jnp.concatenate([dst, jnp.full((pad,), _N, i32)])
    typp = jnp.concatenate([typ, jnp.zeros((pad,), i32)])
    wcat = [jnp.concatenate(
        [V[l].transpose(1, 0, 2).reshape(_D, _BD), W_self[l]], axis=1)
        for l in range(_L)]
    brow = [bias[l].reshape(1, _D) for l in range(_L)]
    zrows = jnp.zeros((_ZR, _D), f32)

    y0, s0 = _mm_first(x, wcat[0], brow[0])
    ag = _edge_kernel(y0, srcp, dstp, typp, A[0], zrows)
    h1, y1, s1 = _mm_mid(ag[:_N], ag[_NPE:_NPE + _N], s0, wcat[1], brow[1])
    ag = _edge_kernel(y1, srcp, dstp, typp, A[1], zrows)
    h2, y2, s2 = _mm_mid(ag[:_N], ag[_NPE:_NPE + _N], s1, wcat[2], brow[2])
    ag = _edge_kernel(y2, srcp, dstp, typp, A[2], zrows)
    h3 = _mm_last(ag[:_N], ag[_NPE:_NPE + _N], s2)

    npad = jnp.zeros((_NP2 - _N, _D), f32)
    h1p = jnp.concatenate([h1, npad], axis=0)
    h2p = jnp.concatenate([h2, npad], axis=0)
    h3p = jnp.concatenate([h3, npad], axis=0)
    gidp = jnp.concatenate(
        [graph_ids.astype(i32), jnp.full((_NP2 - _N,), _B, i32)])
    zsum = jnp.zeros((_GZR, _D), f32)
    zcnt = jnp.zeros((_GZR, _D), f32)
    ones_i = jnp.ones((_PCH, _D), f32)

    pooled = _pool_kernel(
        h1p, h2p, h3p, gidp, head_ids.astype(i32), tail_ids.astype(i32),
        rel_labels.astype(i32), rel_emb, zsum, zcnt, ones_i)

    ws = [fcW[k * _D:(k + 1) * _D, 0].reshape(1, _D) for k in range(10)]
    fb = fcb.reshape(1, 1)
    return _final(*pooled, *ws, fb)


# speedup vs baseline: 1.6343x; 1.0634x over previous
import functools

import jax
import jax.numpy as jnp
from jax import lax
from jax.experimental import pallas as pl
from jax.experimental.pallas import tpu as pltpu
from jax.experimental.pallas import tpu_sc as plsc

_N, _D, _L, _NB, _NR, _B, _E = 10000, 128, 3, 4, 64, 512, 160000
_BD = _NB * _D
_BDP = _BD // 2

_NC, _NS, _LN = 2, 16, 16
_NW = _NC * _NS

_EP = 163840
_EPW = _EP // _NW
_ECH = 64
_ENCH = _EPW // _ECH
_NPE = 10112
_ZR = _NPE // _NS

_NP2 = 10240
_RPW = _NP2 // _NW
_PCH = 64
_PNCH = _RPW // _PCH
_GP = 640
_GZR = _GP // _NS
_BW = _B // _NW

_RB = 400


def _mesh():
    return plsc.VectorSubcoreMesh(core_axis_name="c", subcore_axis_name="s")


def _pack_y(p):
    u = lax.bitcast_convert_type(p.astype(jnp.bfloat16), jnp.uint16)
    u = u.astype(jnp.uint32)
    p01 = u[:, 0 * _D:1 * _D] | (u[:, 1 * _D:2 * _D] << 16)
    p23 = u[:, 2 * _D:3 * _D] | (u[:, 3 * _D:4 * _D] << 16)
    return lax.bitcast_convert_type(
        jnp.concatenate([p01, p23], axis=1), jnp.int32)


def _mm_first_body(x_ref, w_ref, b_ref, y_ref, s_ref):
    p = jnp.dot(x_ref[...], w_ref[...], preferred_element_type=jnp.float32)
    y_ref[...] = _pack_y(p[:, :_BD])
    s_ref[...] = p[:, _BD:] + b_ref[...]


def _mm_first(x, wcat, b):
    return pl.pallas_call(
        _mm_first_body,
        grid=(_N // _RB,),
        in_specs=[
            pl.BlockSpec((_RB, _D), lambda i: (i, 0)),
            pl.BlockSpec((_D, _BD + _D), lambda i: (0, 0)),
            pl.BlockSpec((1, _D), lambda i: (0, 0)),
        ],
        out_specs=[
            pl.BlockSpec((_RB, _BDP), lambda i: (i, 0)),
            pl.BlockSpec((_RB, _D), lambda i: (i, 0)),
        ],
        out_shape=[
            jax.ShapeDtypeStruct((_N, _BDP), jnp.int32),
            jax.ShapeDtypeStruct((_N, _D), jnp.float32),
        ],
    )(x, wcat, b)


def _mm_mid_body(a0_ref, a1_ref, sp_ref, w_ref, b_ref, h_ref, y_ref, s_ref):
    h = jnp.maximum(a0_ref[...] + a1_ref[...] + sp_ref[...], 0.0)
    h_ref[...] = h
    p = jnp.dot(h, w_ref[...], preferred_element_type=jnp.float32)
    y_ref[...] = _pack_y(p[:, :_BD])
    s_ref[...] = p[:, _BD:] + b_ref[...]


def _mm_mid(a0, a1, sp, wcat, b):
    return pl.pallas_call(
        _mm_mid_body,
        grid=(_N // _RB,),
        in_specs=[
            pl.BlockSpec((_RB, _D), lambda i: (i, 0)),
            pl.BlockSpec((_RB, _D), lambda i: (i, 0)),
            pl.BlockSpec((_RB, _D), lambda i: (i, 0)),
            pl.BlockSpec((_D, _BD + _D), lambda i: (0, 0)),
            pl.BlockSpec((1, _D), lambda i: (0, 0)),
        ],
        out_specs=[
            pl.BlockSpec((_RB, _D), lambda i: (i, 0)),
            pl.BlockSpec((_RB, _BDP), lambda i: (i, 0)),
            pl.BlockSpec((_RB, _D), lambda i: (i, 0)),
        ],
        out_shape=[
            jax.ShapeDtypeStruct((_N, _D), jnp.float32),
            jax.ShapeDtypeStruct((_N, _BDP), jnp.int32),
            jax.ShapeDtypeStruct((_N, _D), jnp.float32),
        ],
    )(a0, a1, sp, wcat, b)


def _mm_last_body(a0_ref, a1_ref, sp_ref, h_ref):
    h_ref[...] = jnp.maximum(a0_ref[...] + a1_ref[...] + sp_ref[...], 0.0)


def _mm_last(a0, a1, sp):
    return pl.pallas_call(
        _mm_last_body,
        grid=(_N // _RB,),
        in_specs=[
            pl.BlockSpec((_RB, _D), lambda i: (i, 0)),
            pl.BlockSpec((_RB, _D), lambda i: (i, 0)),
            pl.BlockSpec((_RB, _D), lambda i: (i, 0)),
        ],
        out_specs=pl.BlockSpec((_RB, _D), lambda i: (i, 0)),
        out_shape=jax.ShapeDtypeStruct((_N, _D), jnp.float32),
    )(a0, a1, sp)


def _final_body(s1, s2, s3, c, h1, h2, h3, t1, t2, t3, r,
                wg1, wg2, wg3, wh1, wh2, wh3, wt1, wt2, wt3, wr, fb, o_ref):
    cnt = jnp.maximum(c[:_B, :1] + c[_B:, :1], 1.0)
    acc = jnp.sum(((s1[:_B] + s1[_B:]) / cnt) * wg1[...], axis=1,
                  keepdims=True)
    acc = acc + jnp.sum(((s2[:_B] + s2[_B:]) / cnt) * wg2[...], axis=1,
                        keepdims=True)
    acc = acc + jnp.sum(((s3[:_B] + s3[_B:]) / cnt) * wg3[...], axis=1,
                        keepdims=True)
    for hh, ww in ((h1, wh1), (h2, wh2), (h3, wh3),
                   (t1, wt1), (t2, wt2), (t3, wt3), (r, wr)):
        acc = acc + jnp.sum(hh[...] * ww[...], axis=1, keepdims=True)
    o_ref[...] = acc + fb[...]


def _final(*args):
    return pl.pallas_call(
        _final_body,
        out_shape=jax.ShapeDtypeStruct((_B, 1), jnp.float32),
    )(*args)


def _edge_kw():
    return dict(
        out_type=jax.ShapeDtypeStruct((2 * _NPE, _D), jnp.float32),
        mesh=_mesh(),
        compiler_params=pltpu.CompilerParams(needs_layout_passes=False),
        scratch_types=[
            pltpu.VMEM((_ECH,), jnp.int32),
            pltpu.VMEM((_ECH,), jnp.int32),
            pltpu.VMEM((_ECH,), jnp.int32),
            pltpu.VMEM((_ECH,), jnp.int32),
            pltpu.VMEM((_ECH,), jnp.int32),
            pltpu.VMEM((_ECH,), jnp.int32),
            pltpu.VMEM((_NR, _NB), jnp.float32),
            pltpu.VMEM((_ECH, _BDP), jnp.int32),
            pltpu.VMEM((_ECH, _BDP), jnp.int32),
            pltpu.VMEM((_ECH, _D), jnp.float32),
            pltpu.VMEM_SHARED((_NPE, _D), jnp.float32),
            pltpu.SemaphoreType.DMA,
            pltpu.SemaphoreType.DMA,
        ],
    )


def _edge_body(y, srcp, dstp, typp, a_l, zrows, out,
               s0v, d0v, t0v, s1v, d1v, t1v,
               at_v, g0, g1, m0, agg_sh, sem0, sem1):
    cid = lax.axis_index("c")
    sid = lax.axis_index("s")
    wid = sid * _NC + cid
    rows_slice = pl.ds(sid * _ZR, _ZR)
    pltpu.sync_copy(zrows, agg_sh.at[rows_slice])
    pltpu.sync_copy(a_l, at_v)
    plsc.subcore_barrier()

    lane = jnp.arange(_LN, dtype=jnp.int32)
    wbase = wid * _EPW

    def stage(c, sv, dv, tv):
        base = wbase + c * _ECH
        pltpu.sync_copy(srcp.at[pl.ds(base, _ECH)], sv)
        pltpu.sync_copy(dstp.at[pl.ds(base, _ECH)], dv)
        pltpu.sync_copy(typp.at[pl.ds(base, _ECH)], tv)

    def compute(tv, dv, g, m):
        for j in range(_ECH // _LN):
            typ16 = tv[pl.ds(j * _LN, _LN)]
            rows = jnp.full((_LN,), j * _LN, jnp.int32) + lane
            coefs = [
                plsc.load_gather(at_v, [typ16, jnp.full((_LN,), b, jnp.int32)])
                for b in range(_NB)
            ]
            cb1 = jnp.full((_LN,), _D, jnp.int32)
            mask = jnp.full((_LN,), -65536, jnp.int32)

            def unpack(v):
                lo = lax.bitcast_convert_type(v << 16, jnp.float32)
                hi = lax.bitcast_convert_type(v & mask, jnp.float32)
                return lo, hi

            def qbody(q, c2):
                d0 = q * 4
                for k in range(4):
                    dd = d0 + k
                    i0 = jnp.full((_LN,), dd, jnp.int32)
                    f0, f1 = unpack(plsc.load_gather(g, [rows, i0]))
                    f2, f3 = unpack(plsc.load_gather(g, [rows, cb1 + dd]))
                    acc = (coefs[0] * f0 + coefs[1] * f1
                           + coefs[2] * f2 + coefs[3] * f3)
                    plsc.store_scatter(m, [rows, i0], acc)
                return c2

            lax.fori_loop(0, _D // 4, qbody, 0)
        pltpu.sync_copy(m, agg_sh.at[dv], add=True)

    stage(0, s0v, d0v, t0v)
    pltpu.async_copy(y.at[s0v], g0, sem0)

    def step(s, carry):
        c0 = 2 * s
        stage(c0 + 1, s1v, d1v, t1v)
        pltpu.async_copy(y.at[s1v], g1, sem1)
        pltpu.make_async_copy(y.at[s0v], g0, sem0).wait()
        compute(t0v, d0v, g0, m0)
        stage(c0 + 2, s0v, d0v, t0v)
        pltpu.async_copy(y.at[s0v], g0, sem0)
        pltpu.make_async_copy(y.at[s1v], g1, sem1).wait()
        compute(t1v, d1v, g1, m0)
        return carry

    lax.fori_loop(0, _ENCH // 2, step, 0)
    pltpu.make_async_copy(y.at[s0v], g0, sem0).wait()

    plsc.subcore_barrier()
    pltpu.sync_copy(agg_sh.at[rows_slice],
                    out.at[pl.ds(cid * _NPE + sid * _ZR, _ZR)])


_edge_kernel = pl.kernel(_edge_body, **_edge_kw())


def _pool_kw():
    return dict(out_type=(
        [jax.ShapeDtypeStruct((2 * _B, _D), jnp.float32)] * _L
        + [jax.ShapeDtypeStruct((2 * _B, _D), jnp.float32)]
        + [jax.ShapeDtypeStruct((_B, _D), jnp.float32)] * _L
        + [jax.ShapeDtypeStruct((_B, _D), jnp.float32)] * _L
        + [jax.ShapeDtypeStruct((_B, _D), jnp.float32)]
    ),
        mesh=_mesh(),
        compiler_params=pltpu.CompilerParams(needs_layout_passes=False),
        scratch_types=[
            pltpu.VMEM((_PCH,), jnp.int32),
            pltpu.VMEM((_PCH, _D), jnp.float32),
            pltpu.VMEM((_PCH, _D), jnp.float32),
            pltpu.VMEM((_BW,), jnp.int32),
            pltpu.VMEM((_BW, _D), jnp.float32),
            pltpu.VMEM_SHARED((_GP, _D), jnp.float32),
            pltpu.VMEM_SHARED((_GP, _D), jnp.float32),
            pltpu.VMEM_SHARED((_GP, _D), jnp.float32),
            pltpu.VMEM_SHARED((_GP, _D), jnp.float32),
            pltpu.SemaphoreType.DMA,
        ],
    )


def _pool_body(h1, h2, h3, gid, head_i, tail_i, rlab_i, remb,
               zsum, zcnt, ones_i,
               s1o, s2o, s3o, co, h1o, h2o, h3o, t1o, t2o, t3o, ro,
               gidx_v, row_v, ones_v, bidx_v, gbuf_v,
               ss1, ss2, ss3, scnt_sh, sem):
    cid = lax.axis_index("c")
    sid = lax.axis_index("s")
    wid = sid * _NC + cid
    zs = pl.ds(sid * _GZR, _GZR)
    pltpu.sync_copy(zsum, ss1.at[zs])
    pltpu.sync_copy(zsum, ss2.at[zs])
    pltpu.sync_copy(zsum, ss3.at[zs])
    pltpu.sync_copy(zcnt, scnt_sh.at[zs])
    pltpu.sync_copy(ones_i, ones_v)
    plsc.subcore_barrier()

    def chunk(i, carry):
        base = wid * _RPW + i * _PCH
        sl = pl.ds(base, _PCH)
        pltpu.sync_copy(gid.at[sl], gidx_v)
        pltpu.sync_copy(h1.at[sl], row_v)
        pltpu.sync_copy(row_v, ss1.at[gidx_v], add=True)
        pltpu.sync_copy(h2.at[sl], row_v)
        pltpu.sync_copy(row_v, ss2.at[gidx_v], add=True)
        pltpu.sync_copy(h3.at[sl], row_v)
        pltpu.sync_copy(row_v, ss3.at[gidx_v], add=True)
        pltpu.sync_copy(ones_v, scnt_sh.at[gidx_v], add=True)
        return carry

    lax.fori_loop(0, _PNCH, chunk, 0)
    plsc.subcore_barrier()

    dump = pl.ds(sid * (_B // _NS), _B // _NS)
    dump_o = pl.ds(cid * _B + sid * (_B // _NS), _B // _NS)
    pltpu.sync_copy(ss1.at[dump], s1o.at[dump_o])
    pltpu.sync_copy(ss2.at[dump], s2o.at[dump_o])
    pltpu.sync_copy(ss3.at[dump], s3o.at[dump_o])
    pltpu.sync_copy(scnt_sh.at[dump], co.at[dump_o])

    b0 = pl.ds(wid * _BW, _BW)
    pltpu.sync_copy(head_i.at[b0], bidx_v)
    pltpu.async_copy(h1.at[bidx_v], gbuf_v, sem).wait()
    pltpu.sync_copy(gbuf_v, h1o.at[b0])
    pltpu.async_copy(h2.at[bidx_v], gbuf_v, sem).wait()
    pltpu.sync_copy(gbuf_v, h2o.at[b0])
    pltpu.async_copy(h3.at[bidx_v], gbuf_v, sem).wait()
    pltpu.sync_copy(gbuf_v, h3o.at[b0])
    pltpu.sync_copy(tail_i.at[b0], bidx_v)
    pltpu.async_copy(h1.at[bidx_v], gbuf_v, sem).wait()
    pltpu.sync_copy(gbuf_v, t1o.at[b0])
    pltpu.async_copy(h2.at[bidx_v], gbuf_v, sem).wait()
    pltpu.sync_copy(gbuf_v, t2o.at[b0])
    pltpu.async_copy(h3.at[bidx_v], gbuf_v, sem).wait()
    pltpu.sync_copy(gbuf_v, t3o.at[b0])
    pltpu.sync_copy(rlab_i.at[b0], bidx_v)
    pltpu.async_copy(remb.at[bidx_v], gbuf_v, sem).wait()
    pltpu.sync_copy(gbuf_v, ro.at[b0])


_pool_kernel = pl.kernel(_pool_body, **_pool_kw())


def kernel(x, edge_index, edge_type, graph_ids, head_ids, tail_ids,
           rel_labels, V, A, W_self, bias, rel_emb, fcW, fcb):
    f32 = jnp.float32
    i32 = jnp.int32
    src = edge_index[0].astype(i32)
    dst = edge_index[1].astype(i32)
    typ = edge_type.astype(i32)
    pad = _EP + _ECH - _E
    srcp = jnp.concatenate([src, jnp.zeros((pad,), i32)])
    dstp = jnp.concatenate([dst, jnp.full((pad,), _N, i32)])
    typp = jnp.concatenate([typ, jnp.zeros((pad,), i32)])
    wcat = [jnp.concatenate(
        [V[l].transpose(1, 0, 2).reshape(_D, _BD), W_self[l]], axis=1)
        for l in range(_L)]
    brow = [bias[l].reshape(1, _D) for l in range(_L)]
    zrows = jnp.zeros((_ZR, _D), f32)

    y0, s0 = _mm_first(x, wcat[0], brow[0])
    ag = _edge_kernel(y0, srcp, dstp, typp, A[0], zrows)
    h1, y1, s1 = _mm_mid(ag[:_N], ag[_NPE:_NPE + _N], s0, wcat[1], brow[1])
    ag = _edge_kernel(y1, srcp, dstp, typp, A[1], zrows)
    h2, y2, s2 = _mm_mid(ag[:_N], ag[_NPE:_NPE + _N], s1, wcat[2], brow[2])
    ag = _edge_kernel(y2, srcp, dstp, typp, A[2], zrows)
    h3 = _mm_last(ag[:_N], ag[_NPE:_NPE + _N], s2)

    npad = jnp.zeros((_NP2 - _N, _D), f32)
    h1p = jnp.concatenate([h1, npad], axis=0)
    h2p = jnp.concatenate([h2, npad], axis=0)
    h3p = jnp.concatenate([h3, npad], axis=0)
    gidp = jnp.concatenate(
        [graph_ids.astype(i32), jnp.full((_NP2 - _N,), _B, i32)])
    zsum = jnp.zeros((_GZR, _D), f32)
    zcnt = jnp.zeros((_GZR, _D), f32)
    ones_i = jnp.ones((_PCH, _D), f32)

    pooled = _pool_kernel(
        h1p, h2p, h3p, gidp, head_ids.astype(i32), tail_ids.astype(i32),
        rel_labels.astype(i32), rel_emb, zsum, zcnt, ones_i)

    ws = [fcW[k * _D:(k + 1) * _D, 0].reshape(1, _D) for k in range(10)]
    fb = fcb.reshape(1, 1)
    return _final(*pooled, *ws, fb)
